# baseline (speedup 1.0000x reference)
import jax
import jax.numpy as jnp
from jax.experimental import pallas as pl
from jax.experimental.pallas import tpu as pltpu

HK = 4
HV = 8
DK = 128
DV = 128
KW = 4
QKV = DK * HK * 2 + DV * HV
SSM = HV * DV * DK
CONV = (KW - 1) * QKV
BLOCK = SSM + CONV
NB = 512
B = 128
HSZ = DV * DK


def _sigmoid(x):
    return 1.0 / (1.0 + jnp.exp(-x))


def _softplus(x):
    return jnp.maximum(x, 0.0) + jnp.log(1.0 + jnp.exp(-jnp.abs(x)))


def _gate(arow, brow, dtb, alog):
    g = -jnp.exp(alog) * _softplus(arow + dtb)
    beta = _sigmoid(brow)
    return g, beta


def _conv_x(c0, c1, c2, mq, cw_ref):
    co = (c0 * cw_ref[0:1, :] + c1 * cw_ref[1:2, :]
          + c2 * cw_ref[2:3, :] + mq * cw_ref[3:4, :])
    return co * _sigmoid(co)


def _l2n(t):
    return t * jax.lax.rsqrt(jnp.sum(t * t, axis=1, keepdims=True) + 1e-6)


_IOTA = None


def _lane_iota():
    return jax.lax.broadcasted_iota(jnp.int32, (8, 128), 1)


def _head_loop(src_load, store, K, Q, V, EGcol, Bcol, with_o):
    iota = _lane_iota()

    def body(v, O8):
        Svec = src_load(v)
        Sg = Svec * EGcol
        kv = jnp.sum(Sg * K, axis=1, keepdims=True)
        maskv = (iota == v).astype(jnp.float32)
        Vcol = jnp.sum(V * maskv, axis=1, keepdims=True)
        Dcol = (Vcol - kv) * Bcol
        Snew = Sg + Dcol * K
        if store is not None:
            store(v, Snew)
        if with_o:
            onum = jnp.sum(Snew * Q, axis=1, keepdims=True)
            O8 = O8 + onum * maskv
        return O8

    return jax.lax.fori_loop(0, DV, body, jnp.zeros((8, 128), jnp.float32))


def _mem_kernel(tfb_ref, mem_ref, mq_ref, b_ref, a_ref, dtb_ref, alog_ref,
                cw_ref, out_ref):
    n = pl.program_id(0)
    toks = [tfb_ref[8 * n + r] for r in range(8)]
    nvalid = sum([(t >= 0).astype(jnp.int32) for t in toks])

    @pl.when(nvalid == 0)
    def _copy():
        out_ref[...] = mem_ref[...]

    @pl.when(nvalid > 0)
    def _update():
        dtb = dtb_ref[:]
        alog = alog_ref[:]
        mqs, kns, vrs, egs, bts, vfs = [], [], [], [], [], []
        for r in range(8):
            tok = toks[r]
            valid = tok >= 0
            tk = jnp.maximum(tok, 0)
            mqrow = mq_ref[pl.ds(tk, 1), :]
            x = _conv_x(mem_ref[r:r + 1, SSM:SSM + QKV],
                        mem_ref[r:r + 1, SSM + QKV:SSM + 2 * QKV],
                        mem_ref[r:r + 1, SSM + 2 * QKV:SSM + 3 * QKV],
                        mqrow, cw_ref)
            g, beta = _gate(a_ref[pl.ds(tk, 1), :], b_ref[pl.ds(tk, 1), :],
                            dtb, alog)
            vf = valid.astype(jnp.float32)
            egs.append(jnp.where(valid, jnp.exp(g), 1.0))
            bts.append(beta * vf)
            kns.append([_l2n(x[:, (HK + h) * DK:(HK + h + 1) * DK])
                        for h in range(HK)])
            vrs.append([x[:, (2 * HK + hh) * DV:(2 * HK + hh + 1) * DV]
                        for hh in range(HV)])
            mqs.append(mqrow)
            vfs.append(jnp.full((1, 1), vf))
        EG8 = jnp.concatenate(egs, axis=0)
        BT8 = jnp.concatenate(bts, axis=0)
        K4 = [jnp.concatenate([kns[r][h] for r in range(8)], axis=0)
              for h in range(HK)]
        V8 = [jnp.concatenate([vrs[r][hh] for r in range(8)], axis=0)
              for hh in range(HV)]
        vcol = jnp.concatenate(vfs, axis=0)

        for hh in range(HV):
            base = hh * HSZ
            _head_loop(
                lambda v, base=base: mem_ref[:, pl.ds(base + v * DK, DK)],
                lambda v, val, base=base: out_ref.__setitem__(
                    (slice(None), pl.ds(base + v * DK, DK)), val),
                K4[hh // 2], None, V8[hh],
                EG8[:, hh:hh + 1], BT8[:, hh:hh + 1], with_o=False)

        shifted = mem_ref[:, SSM + QKV:SSM + 3 * QKV]
        orig01 = mem_ref[:, SSM:SSM + 2 * QKV]
        out_ref[:, SSM:SSM + 2 * QKV] = vcol * shifted + (1.0 - vcol) * orig01
        Mq8 = jnp.concatenate(mqs, axis=0)
        orig2 = mem_ref[:, SSM + 2 * QKV:SSM + 3 * QKV]
        out_ref[:, SSM + 2 * QKV:SSM + 3 * QKV] = (
            vcol * Mq8 + (1.0 - vcol) * orig2)


def _o_kernel(bi_ref, kv_ref, mq_ref, b_ref, a_ref, dtb_ref, alog_ref,
              cw_ref, o_ref, scr, sems):
    j = pl.program_id(0)
    copies = []
    for r in range(8):
        bi = bi_ref[8 * j + r]
        cp = pltpu.make_async_copy(kv_ref.at[pl.ds(bi, 1), :],
                                   scr.at[pl.ds(r, 1), :], sems.at[r])
        cp.start()
        copies.append(cp)
    for cp in copies:
        cp.wait()

    x = _conv_x(scr[:, SSM:SSM + QKV],
                scr[:, SSM + QKV:SSM + 2 * QKV],
                scr[:, SSM + 2 * QKV:SSM + 3 * QKV],
                mq_ref[...], cw_ref)
    g, beta = _gate(a_ref[...], b_ref[...], dtb_ref[:], alog_ref[:])
    for hh in range(HV):
        h = hh // 2
        K = _l2n(x[:, (HK + h) * DK:(HK + h + 1) * DK])
        Q = _l2n(x[:, h * DK:(h + 1) * DK]) * (DK ** -0.5)
        V = x[:, (2 * HK + hh) * DV:(2 * HK + hh + 1) * DV]
        base = hh * HSZ
        O8 = _head_loop(
            lambda v, base=base: scr[:, pl.ds(base + v * DK, DK)],
            None, K, Q, V,
            jnp.exp(g[:, hh:hh + 1]), beta[:, hh:hh + 1], with_o=True)
        o_ref[:, hh * DV:(hh + 1) * DV] = O8


def kernel(mixed_qkv, b, a, kv_cache_mem, conv_weights, dt_bias, alog,
           block_idx):
    f32 = jnp.float32
    cw = conv_weights.T
    dtb2 = dt_bias.reshape(1, HV)
    alog2 = alog.reshape(1, HV)
    tfb = jnp.full((NB,), -1, jnp.int32).at[block_idx].set(
        jnp.arange(B, dtype=jnp.int32))

    new_mem = pl.pallas_call(
        _mem_kernel,
        grid_spec=pltpu.PrefetchScalarGridSpec(
            num_scalar_prefetch=1,
            grid=(NB // 8,),
            in_specs=[
                pl.BlockSpec((8, BLOCK), lambda n, s: (n, 0)),
                pl.BlockSpec((B, QKV), lambda n, s: (0, 0)),
                pl.BlockSpec((B, HV), lambda n, s: (0, 0)),
                pl.BlockSpec((B, HV), lambda n, s: (0, 0)),
                pl.BlockSpec((1, HV), lambda n, s: (0, 0)),
                pl.BlockSpec((1, HV), lambda n, s: (0, 0)),
                pl.BlockSpec((KW, QKV), lambda n, s: (0, 0)),
            ],
            out_specs=pl.BlockSpec((8, BLOCK), lambda n, s: (n, 0)),
        ),
        out_shape=jax.ShapeDtypeStruct((NB, BLOCK), f32),
    )(tfb, kv_cache_mem, mixed_qkv, b, a, dtb2, alog2, cw)

    o_flat = pl.pallas_call(
        _o_kernel,
        grid_spec=pltpu.PrefetchScalarGridSpec(
            num_scalar_prefetch=1,
            grid=(B // 8,),
            in_specs=[
                pl.BlockSpec(memory_space=pl.ANY),
                pl.BlockSpec((8, QKV), lambda j, s: (j, 0)),
                pl.BlockSpec((8, HV), lambda j, s: (j, 0)),
                pl.BlockSpec((8, HV), lambda j, s: (j, 0)),
                pl.BlockSpec((1, HV), lambda j, s: (0, 0)),
                pl.BlockSpec((1, HV), lambda j, s: (0, 0)),
                pl.BlockSpec((KW, QKV), lambda j, s: (0, 0)),
            ],
            out_specs=pl.BlockSpec((8, HV * DV), lambda j, s: (j, 0)),
            scratch_shapes=[
                pltpu.VMEM((8, BLOCK), f32),
                pltpu.SemaphoreType.DMA((8,)),
            ],
        ),
        out_shape=jax.ShapeDtypeStruct((B, HV * DV), f32),
    )(block_idx, kv_cache_mem, mixed_qkv, b, a, dtb2, alog2, cw)

    return o_flat.reshape(B, HV, DV), new_mem


# speedup vs baseline: 39.2777x; 39.2777x over previous
import jax
import jax.numpy as jnp
from jax.experimental import pallas as pl
from jax.experimental.pallas import tpu as pltpu

HK = 4
HV = 8
DK = 128
DV = 128
KW = 4
QKV = DK * HK * 2 + DV * HV
SSM = HV * DV * DK
CONV = (KW - 1) * QKV
BLOCK = SSM + CONV
NB = 512
B = 128
HSZ = DV * DK


def _sigmoid(x):
    return 1.0 / (1.0 + jnp.exp(-x))


def _softplus(x):
    return jnp.maximum(x, 0.0) + jnp.log(1.0 + jnp.exp(-jnp.abs(x)))


def _gate(arow, brow, dtb, alog):
    g = -jnp.exp(alog) * _softplus(arow + dtb)
    beta = _sigmoid(brow)
    return g, beta


def _conv_x(c0, c1, c2, mq, cw_ref):
    co = (c0 * cw_ref[0:1, :] + c1 * cw_ref[1:2, :]
          + c2 * cw_ref[2:3, :] + mq * cw_ref[3:4, :])
    return co * _sigmoid(co)


def _l2n(t):
    return t * jax.lax.rsqrt(jnp.sum(t * t, axis=1, keepdims=True) + 1e-6)


SPAN = 4096
NSEG = SPAN // DK
NSPAN = HSZ // SPAN

_HIGH = jax.lax.Precision.DEFAULT


def _seg_matrix():
    rio = jax.lax.broadcasted_iota(jnp.int32, (NSEG, SPAN), 0)
    lio = jax.lax.broadcasted_iota(jnp.int32, (NSEG, SPAN), 1)
    return ((lio // DK) == rio).astype(jnp.float32)


def _seg_reduce(t, E):
    return jax.lax.dot_general(t, E, (((1,), (1,)), ((), ())),
                               precision=_HIGH)


def _seg_bcast(d, E):
    return jax.lax.dot_general(d, E, (((1,), (0,)), ((), ())),
                               precision=_HIGH)


def _mem_kernel(tfb_ref, mem_ref, mq_ref, b_ref, a_ref, dtb_ref, alog_ref,
                cw_ref, out_ref):
    n = pl.program_id(0)
    toks = [tfb_ref[8 * n + r] for r in range(8)]
    nvalid = sum([(t >= 0).astype(jnp.int32) for t in toks])

    @pl.when(nvalid == 0)
    def _copy():
        out_ref[...] = mem_ref[...]

    @pl.when(nvalid > 0)
    def _update():
        dtb = dtb_ref[:]
        alog = alog_ref[:]
        mqs, kns, vrs, egs, bts, vfs = [], [], [], [], [], []
        for r in range(8):
            tok = toks[r]
            valid = tok >= 0
            tk = jnp.maximum(tok, 0)
            mqrow = mq_ref[pl.ds(tk, 1), :]
            x = _conv_x(mem_ref[r:r + 1, SSM:SSM + QKV],
                        mem_ref[r:r + 1, SSM + QKV:SSM + 2 * QKV],
                        mem_ref[r:r + 1, SSM + 2 * QKV:SSM + 3 * QKV],
                        mqrow, cw_ref)
            g, beta = _gate(a_ref[pl.ds(tk, 1), :], b_ref[pl.ds(tk, 1), :],
                            dtb, alog)
            vf = valid.astype(jnp.float32)
            egs.append(jnp.where(valid, jnp.exp(g), 1.0))
            bts.append(beta * vf)
            kns.append([_l2n(x[:, (HK + h) * DK:(HK + h + 1) * DK])
                        for h in range(HK)])
            vrs.append([x[:, (2 * HK + hh) * DV:(2 * HK + hh + 1) * DV]
                        for hh in range(HV)])
            mqs.append(mqrow)
            vfs.append(jnp.full((1, 1), vf))
        EG8 = jnp.concatenate(egs, axis=0)
        BT8 = jnp.concatenate(bts, axis=0)
        K4 = [jnp.concatenate([kns[r][h] for r in range(8)], axis=0)
              for h in range(HK)]
        V8 = [jnp.concatenate([vrs[r][hh] for r in range(8)], axis=0)
              for hh in range(HV)]
        vcol = jnp.concatenate(vfs, axis=0)

        E = _seg_matrix()
        for hh in range(HV):
            base = hh * HSZ
            K = K4[hh // 2]
            Kt = jnp.concatenate([K] * NSEG, axis=1)
            EGc = EG8[:, hh:hh + 1]
            kvs = []
            for i in range(NSPAN):
                W = mem_ref[:, base + i * SPAN:base + (i + 1) * SPAN]
                kvs.append(_seg_reduce((W * EGc) * Kt, E))
            KVm = jnp.concatenate(kvs, axis=1)
            Dm = (V8[hh] - KVm) * BT8[:, hh:hh + 1]
            for i in range(NSPAN):
                W = mem_ref[:, base + i * SPAN:base + (i + 1) * SPAN]
                Dx = _seg_bcast(Dm[:, NSEG * i:NSEG * (i + 1)], E)
                out_ref[:, base + i * SPAN:base + (i + 1) * SPAN] = (
                    W * EGc + Dx * Kt)

        shifted = mem_ref[:, SSM + QKV:SSM + 3 * QKV]
        orig01 = mem_ref[:, SSM:SSM + 2 * QKV]
        out_ref[:, SSM:SSM + 2 * QKV] = vcol * shifted + (1.0 - vcol) * orig01
        Mq8 = jnp.concatenate(mqs, axis=0)
        orig2 = mem_ref[:, SSM + 2 * QKV:SSM + 3 * QKV]
        out_ref[:, SSM + 2 * QKV:SSM + 3 * QKV] = (
            vcol * Mq8 + (1.0 - vcol) * orig2)


def _o_kernel(bi_ref, kv_ref, mq_ref, b_ref, a_ref, dtb_ref, alog_ref,
              cw_ref, o_ref, scr, sems):
    j = pl.program_id(0)
    copies = []
    for r in range(8):
        bi = bi_ref[8 * j + r]
        cp = pltpu.make_async_copy(kv_ref.at[pl.ds(bi, 1), :],
                                   scr.at[pl.ds(r, 1), :], sems.at[r])
        cp.start()
        copies.append(cp)
    for cp in copies:
        cp.wait()

    x = _conv_x(scr[:, SSM:SSM + QKV],
                scr[:, SSM + QKV:SSM + 2 * QKV],
                scr[:, SSM + 2 * QKV:SSM + 3 * QKV],
                mq_ref[...], cw_ref)
    g, beta = _gate(a_ref[...], b_ref[...], dtb_ref[:], alog_ref[:])
    E = _seg_matrix()
    for hh in range(HV):
        h = hh // 2
        K = _l2n(x[:, (HK + h) * DK:(HK + h + 1) * DK])
        Q = _l2n(x[:, h * DK:(h + 1) * DK]) * (DK ** -0.5)
        V = x[:, (2 * HK + hh) * DV:(2 * HK + hh + 1) * DV]
        base = hh * HSZ
        Kt = jnp.concatenate([K] * NSEG, axis=1)
        Qt = jnp.concatenate([Q] * NSEG, axis=1)
        EGc = jnp.exp(g[:, hh:hh + 1])
        kvs, sqs = [], []
        for i in range(NSPAN):
            Sg = scr[:, base + i * SPAN:base + (i + 1) * SPAN] * EGc
            kvs.append(_seg_reduce(Sg * Kt, E))
            sqs.append(_seg_reduce(Sg * Qt, E))
        KVm = jnp.concatenate(kvs, axis=1)
        SQm = jnp.concatenate(sqs, axis=1)
        Dm = (V - KVm) * beta[:, hh:hh + 1]
        kq = jnp.sum(K * Q, axis=1, keepdims=True)
        o_ref[:, hh * DV:(hh + 1) * DV] = SQm + Dm * kq


def kernel(mixed_qkv, b, a, kv_cache_mem, conv_weights, dt_bias, alog,
           block_idx):
    f32 = jnp.float32
    cw = conv_weights.T
    dtb2 = dt_bias.reshape(1, HV)
    alog2 = alog.reshape(1, HV)
    tfb = jnp.full((NB,), -1, jnp.int32).at[block_idx].set(
        jnp.arange(B, dtype=jnp.int32))

    new_mem = pl.pallas_call(
        _mem_kernel,
        grid_spec=pltpu.PrefetchScalarGridSpec(
            num_scalar_prefetch=1,
            grid=(NB // 8,),
            in_specs=[
                pl.BlockSpec((8, BLOCK), lambda n, s: (n, 0)),
                pl.BlockSpec((B, QKV), lambda n, s: (0, 0)),
                pl.BlockSpec((B, HV), lambda n, s: (0, 0)),
                pl.BlockSpec((B, HV), lambda n, s: (0, 0)),
                pl.BlockSpec((1, HV), lambda n, s: (0, 0)),
                pl.BlockSpec((1, HV), lambda n, s: (0, 0)),
                pl.BlockSpec((KW, QKV), lambda n, s: (0, 0)),
            ],
            out_specs=pl.BlockSpec((8, BLOCK), lambda n, s: (n, 0)),
        ),
        out_shape=jax.ShapeDtypeStruct((NB, BLOCK), f32),
    )(tfb, kv_cache_mem, mixed_qkv, b, a, dtb2, alog2, cw)

    o_flat = pl.pallas_call(
        _o_kernel,
        grid_spec=pltpu.PrefetchScalarGridSpec(
            num_scalar_prefetch=1,
            grid=(B // 8,),
            in_specs=[
                pl.BlockSpec(memory_space=pl.ANY),
                pl.BlockSpec((8, QKV), lambda j, s: (j, 0)),
                pl.BlockSpec((8, HV), lambda j, s: (j, 0)),
                pl.BlockSpec((8, HV), lambda j, s: (j, 0)),
                pl.BlockSpec((1, HV), lambda j, s: (0, 0)),
                pl.BlockSpec((1, HV), lambda j, s: (0, 0)),
                pl.BlockSpec((KW, QKV), lambda j, s: (0, 0)),
            ],
            out_specs=pl.BlockSpec((8, HV * DV), lambda j, s: (j, 0)),
            scratch_shapes=[
                pltpu.VMEM((8, BLOCK), f32),
                pltpu.SemaphoreType.DMA((8,)),
            ],
        ),
        out_shape=jax.ShapeDtypeStruct((B, HV * DV), f32),
    )(block_idx, kv_cache_mem, mixed_qkv, b, a, dtb2, alog2, cw)

    return o_flat.reshape(B, HV, DV), new_mem


# speedup vs baseline: 39.5792x; 1.0077x over previous
import jax
import jax.numpy as jnp
from jax.experimental import pallas as pl
from jax.experimental.pallas import tpu as pltpu

HK = 4
HV = 8
DK = 128
DV = 128
KW = 4
QKV = DK * HK * 2 + DV * HV
SSM = HV * DV * DK
CONV = (KW - 1) * QKV
BLOCK = SSM + CONV
NB = 512
B = 128
HSZ = DV * DK


def _sigmoid(x):
    return 1.0 / (1.0 + jnp.exp(-x))


def _softplus(x):
    return jnp.maximum(x, 0.0) + jnp.log(1.0 + jnp.exp(-jnp.abs(x)))


def _gate(arow, brow, dtb, alog):
    g = -jnp.exp(alog) * _softplus(arow + dtb)
    beta = _sigmoid(brow)
    return g, beta


def _conv_x(c0, c1, c2, mq, cw_ref):
    co = (c0 * cw_ref[0:1, :] + c1 * cw_ref[1:2, :]
          + c2 * cw_ref[2:3, :] + mq * cw_ref[3:4, :])
    return co * _sigmoid(co)


def _l2n(t):
    return t * jax.lax.rsqrt(jnp.sum(t * t, axis=1, keepdims=True) + 1e-6)


SPAN = 4096
NSEG = SPAN // DK
NSPAN = HSZ // SPAN

_HIGH = jax.lax.Precision.DEFAULT


def _seg_matrix():
    rio = jax.lax.broadcasted_iota(jnp.int32, (NSEG, SPAN), 0)
    lio = jax.lax.broadcasted_iota(jnp.int32, (NSEG, SPAN), 1)
    return ((lio // DK) == rio).astype(jnp.float32)


def _seg_reduce(t, E):
    return jax.lax.dot_general(t, E, (((1,), (1,)), ((), ())),
                               precision=_HIGH)


def _seg_bcast(d, E):
    return jax.lax.dot_general(d, E, (((1,), (0,)), ((), ())),
                               precision=_HIGH)


def _mem_kernel(tfb_ref, mem_ref, mq_ref, b_ref, a_ref, dtb_ref, alog_ref,
                cw_ref, out_ref):
    n = pl.program_id(0)
    toks = [tfb_ref[8 * n + r] for r in range(8)]
    nvalid = sum([(t >= 0).astype(jnp.int32) for t in toks])

    @pl.when(nvalid == 0)
    def _copy():
        out_ref[...] = mem_ref[...]

    @pl.when(nvalid > 0)
    def _update():
        mqs, brs, ars, vfs = [], [], [], []
        for r in range(8):
            tok = toks[r]
            tk = jnp.maximum(tok, 0)
            mqs.append(mq_ref[pl.ds(tk, 1), :])
            brs.append(b_ref[pl.ds(tk, 1), :])
            ars.append(a_ref[pl.ds(tk, 1), :])
            vfs.append(jnp.full((1, 1), (tok >= 0).astype(jnp.float32)))
        Mq8 = jnp.concatenate(mqs, axis=0)
        vcol = jnp.concatenate(vfs, axis=0)
        x = _conv_x(mem_ref[:, SSM:SSM + QKV],
                    mem_ref[:, SSM + QKV:SSM + 2 * QKV],
                    mem_ref[:, SSM + 2 * QKV:SSM + 3 * QKV],
                    Mq8, cw_ref)
        g, beta = _gate(jnp.concatenate(ars, axis=0),
                        jnp.concatenate(brs, axis=0),
                        dtb_ref[:], alog_ref[:])
        EG8 = vcol * jnp.exp(g) + (1.0 - vcol)
        BT8 = beta * vcol

        E = _seg_matrix()
        for hh in range(HV):
            base = hh * HSZ
            h = hh // 2
            K = _l2n(x[:, (HK + h) * DK:(HK + h + 1) * DK])
            V = x[:, (2 * HK + hh) * DV:(2 * HK + hh + 1) * DV]
            Kt = jnp.concatenate([K] * NSEG, axis=1)
            EGc = EG8[:, hh:hh + 1]
            kvs = []
            for i in range(NSPAN):
                W = mem_ref[:, base + i * SPAN:base + (i + 1) * SPAN]
                kvs.append(_seg_reduce(W * Kt, E))
            KVm = jnp.concatenate(kvs, axis=1) * EGc
            Dm = (V - KVm) * BT8[:, hh:hh + 1]
            for i in range(NSPAN):
                W = mem_ref[:, base + i * SPAN:base + (i + 1) * SPAN]
                Dx = _seg_bcast(Dm[:, NSEG * i:NSEG * (i + 1)], E)
                out_ref[:, base + i * SPAN:base + (i + 1) * SPAN] = (
                    W * EGc + Dx * Kt)

        shifted = mem_ref[:, SSM + QKV:SSM + 3 * QKV]
        orig01 = mem_ref[:, SSM:SSM + 2 * QKV]
        out_ref[:, SSM:SSM + 2 * QKV] = vcol * shifted + (1.0 - vcol) * orig01
        orig2 = mem_ref[:, SSM + 2 * QKV:SSM + 3 * QKV]
        out_ref[:, SSM + 2 * QKV:SSM + 3 * QKV] = (
            vcol * Mq8 + (1.0 - vcol) * orig2)


def _o_kernel(bi_ref, kv_ref, mq_ref, b_ref, a_ref, dtb_ref, alog_ref,
              cw_ref, o_ref, scr, sems):
    j = pl.program_id(0)
    copies = []
    for r in range(8):
        bi = bi_ref[8 * j + r]
        cp = pltpu.make_async_copy(kv_ref.at[pl.ds(bi, 1), :],
                                   scr.at[pl.ds(r, 1), :], sems.at[r])
        cp.start()
        copies.append(cp)
    for cp in copies:
        cp.wait()

    x = _conv_x(scr[:, SSM:SSM + QKV],
                scr[:, SSM + QKV:SSM + 2 * QKV],
                scr[:, SSM + 2 * QKV:SSM + 3 * QKV],
                mq_ref[...], cw_ref)
    g, beta = _gate(a_ref[...], b_ref[...], dtb_ref[:], alog_ref[:])
    E = _seg_matrix()
    for hh in range(HV):
        h = hh // 2
        K = _l2n(x[:, (HK + h) * DK:(HK + h + 1) * DK])
        Q = _l2n(x[:, h * DK:(h + 1) * DK]) * (DK ** -0.5)
        V = x[:, (2 * HK + hh) * DV:(2 * HK + hh + 1) * DV]
        base = hh * HSZ
        Kt = jnp.concatenate([K] * NSEG, axis=1)
        Qt = jnp.concatenate([Q] * NSEG, axis=1)
        EGc = jnp.exp(g[:, hh:hh + 1])
        kvs, sqs = [], []
        for i in range(NSPAN):
            Sg = scr[:, base + i * SPAN:base + (i + 1) * SPAN] * EGc
            kvs.append(_seg_reduce(Sg * Kt, E))
            sqs.append(_seg_reduce(Sg * Qt, E))
        KVm = jnp.concatenate(kvs, axis=1)
        SQm = jnp.concatenate(sqs, axis=1)
        Dm = (V - KVm) * beta[:, hh:hh + 1]
        kq = jnp.sum(K * Q, axis=1, keepdims=True)
        o_ref[:, hh * DV:(hh + 1) * DV] = SQm + Dm * kq


def kernel(mixed_qkv, b, a, kv_cache_mem, conv_weights, dt_bias, alog,
           block_idx):
    f32 = jnp.float32
    cw = conv_weights.T
    dtb2 = dt_bias.reshape(1, HV)
    alog2 = alog.reshape(1, HV)
    tfb = jnp.full((NB,), -1, jnp.int32).at[block_idx].set(
        jnp.arange(B, dtype=jnp.int32))

    new_mem = pl.pallas_call(
        _mem_kernel,
        grid_spec=pltpu.PrefetchScalarGridSpec(
            num_scalar_prefetch=1,
            grid=(NB // 8,),
            in_specs=[
                pl.BlockSpec((8, BLOCK), lambda n, s: (n, 0)),
                pl.BlockSpec((B, QKV), lambda n, s: (0, 0)),
                pl.BlockSpec((B, HV), lambda n, s: (0, 0)),
                pl.BlockSpec((B, HV), lambda n, s: (0, 0)),
                pl.BlockSpec((1, HV), lambda n, s: (0, 0)),
                pl.BlockSpec((1, HV), lambda n, s: (0, 0)),
                pl.BlockSpec((KW, QKV), lambda n, s: (0, 0)),
            ],
            out_specs=pl.BlockSpec((8, BLOCK), lambda n, s: (n, 0)),
        ),
        out_shape=jax.ShapeDtypeStruct((NB, BLOCK), f32),
    )(tfb, kv_cache_mem, mixed_qkv, b, a, dtb2, alog2, cw)

    o_flat = pl.pallas_call(
        _o_kernel,
        grid_spec=pltpu.PrefetchScalarGridSpec(
            num_scalar_prefetch=1,
            grid=(B // 8,),
            in_specs=[
                pl.BlockSpec(memory_space=pl.ANY),
                pl.BlockSpec((8, QKV), lambda j, s: (j, 0)),
                pl.BlockSpec((8, HV), lambda j, s: (j, 0)),
                pl.BlockSpec((8, HV), lambda j, s: (j, 0)),
                pl.BlockSpec((1, HV), lambda j, s: (0, 0)),
                pl.BlockSpec((1, HV), lambda j, s: (0, 0)),
                pl.BlockSpec((KW, QKV), lambda j, s: (0, 0)),
            ],
            out_specs=pl.BlockSpec((8, HV * DV), lambda j, s: (j, 0)),
            scratch_shapes=[
                pltpu.VMEM((8, BLOCK), f32),
                pltpu.SemaphoreType.DMA((8,)),
            ],
        ),
        out_shape=jax.ShapeDtypeStruct((B, HV * DV), f32),
    )(block_idx, kv_cache_mem, mixed_qkv, b, a, dtb2, alog2, cw)

    return o_flat.reshape(B, HV, DV), new_mem


# speedup vs baseline: 43.2723x; 1.0933x over previous
import jax
import jax.numpy as jnp
from jax.experimental import pallas as pl
from jax.experimental.pallas import tpu as pltpu

HK = 4
HV = 8
DK = 128
DV = 128
KW = 4
QKV = DK * HK * 2 + DV * HV
SSM = HV * DV * DK
CONV = (KW - 1) * QKV
BLOCK = SSM + CONV
NB = 512
B = 128
HSZ = DV * DK


def _sigmoid(x):
    return 1.0 / (1.0 + jnp.exp(-x))


def _softplus(x):
    return jnp.maximum(x, 0.0) + jnp.log(1.0 + jnp.exp(-jnp.abs(x)))


def _gate(arow, brow, dtb, alog):
    g = -jnp.exp(alog) * _softplus(arow + dtb)
    beta = _sigmoid(brow)
    return g, beta


def _conv_x(c0, c1, c2, mq, cw_ref):
    co = (c0 * cw_ref[0:1, :] + c1 * cw_ref[1:2, :]
          + c2 * cw_ref[2:3, :] + mq * cw_ref[3:4, :])
    return co * _sigmoid(co)


def _l2n(t):
    return t * jax.lax.rsqrt(jnp.sum(t * t, axis=1, keepdims=True) + 1e-6)


SPAN = 4096
NSEG = SPAN // DK
NSPAN = HSZ // SPAN

_HIGH = jax.lax.Precision.DEFAULT


def _seg_matrix():
    rio = jax.lax.broadcasted_iota(jnp.int32, (NSEG, SPAN), 0)
    lio = jax.lax.broadcasted_iota(jnp.int32, (NSEG, SPAN), 1)
    return ((lio // DK) == rio).astype(jnp.float32)


def _seg_reduce(t, E):
    return jax.lax.dot_general(t, E, (((1,), (1,)), ((), ())),
                               precision=_HIGH)


def _seg_bcast(d, E):
    return jax.lax.dot_general(d, E, (((1,), (0,)), ((), ())),
                               precision=_HIGH)


def _mem_kernel(tfb_ref, mem_ref, mq_ref, b_ref, a_ref, dtb_ref, alog_ref,
                cw_ref, out_ref):
    n = pl.program_id(0)
    toks = [tfb_ref[8 * n + r] for r in range(8)]
    nvalid = sum([(t >= 0).astype(jnp.int32) for t in toks])

    @pl.when(nvalid == 0)
    def _copy():
        out_ref[...] = mem_ref[...]

    @pl.when(nvalid > 0)
    def _update():
        mqs, brs, ars, vfs = [], [], [], []
        for r in range(8):
            tok = toks[r]
            tk = jnp.maximum(tok, 0)
            mqs.append(mq_ref[pl.ds(tk, 1), :])
            brs.append(b_ref[pl.ds(tk, 1), :])
            ars.append(a_ref[pl.ds(tk, 1), :])
            vfs.append(jnp.full((1, 1), (tok >= 0).astype(jnp.float32)))
        Mq8 = jnp.concatenate(mqs, axis=0)
        vcol = jnp.concatenate(vfs, axis=0)
        x = _conv_x(mem_ref[:, SSM:SSM + QKV],
                    mem_ref[:, SSM + QKV:SSM + 2 * QKV],
                    mem_ref[:, SSM + 2 * QKV:SSM + 3 * QKV],
                    Mq8, cw_ref)
        g, beta = _gate(jnp.concatenate(ars, axis=0),
                        jnp.concatenate(brs, axis=0),
                        dtb_ref[:], alog_ref[:])
        EG8 = vcol * jnp.exp(g) + (1.0 - vcol)
        BT8 = beta * vcol

        E = _seg_matrix()
        for hh in range(HV):
            base = hh * HSZ
            h = hh // 2
            K = _l2n(x[:, (HK + h) * DK:(HK + h + 1) * DK])
            V = x[:, (2 * HK + hh) * DV:(2 * HK + hh + 1) * DV]
            Kt = jnp.concatenate([K] * NSEG, axis=1)
            EGc = EG8[:, hh:hh + 1]
            kvs = []
            for i in range(NSPAN):
                W = mem_ref[:, base + i * SPAN:base + (i + 1) * SPAN]
                kvs.append(_seg_reduce(W * Kt, E))
            KVm = jnp.concatenate(kvs, axis=1) * EGc
            Dm = (V - KVm) * BT8[:, hh:hh + 1]
            for i in range(NSPAN):
                W = mem_ref[:, base + i * SPAN:base + (i + 1) * SPAN]
                Dx = _seg_bcast(Dm[:, NSEG * i:NSEG * (i + 1)], E)
                out_ref[:, base + i * SPAN:base + (i + 1) * SPAN] = (
                    W * EGc + Dx * Kt)

        shifted = mem_ref[:, SSM + QKV:SSM + 3 * QKV]
        orig01 = mem_ref[:, SSM:SSM + 2 * QKV]
        out_ref[:, SSM:SSM + 2 * QKV] = vcol * shifted + (1.0 - vcol) * orig01
        orig2 = mem_ref[:, SSM + 2 * QKV:SSM + 3 * QKV]
        out_ref[:, SSM + 2 * QKV:SSM + 3 * QKV] = (
            vcol * Mq8 + (1.0 - vcol) * orig2)


def _o_kernel(bi_ref, kv_ref, mq_ref, b_ref, a_ref, dtb_ref, alog_ref,
              cw_ref, o_ref, scr, scr1, sems, sems1):
    j = pl.program_id(0)

    def _copies(jj, dst, sem):
        out = []
        for r in range(8):
            bi = bi_ref[8 * jj + r]
            out.append(pltpu.make_async_copy(kv_ref.at[pl.ds(bi, 1), :],
                                             dst.at[pl.ds(r, 1), :],
                                             sem.at[r]))
        return out

    @pl.when(j == 0)
    def _warm():
        for cp in _copies(0, scr, sems):
            cp.start()

    par = jax.lax.rem(j, 2)

    @pl.when(par == 0)
    def _even():
        @pl.when(j + 1 < B // 8)
        def _pre():
            for cp in _copies(j + 1, scr1, sems1):
                cp.start()
        for cp in _copies(j, scr, sems):
            cp.wait()
        _o_compute(scr, mq_ref, b_ref, a_ref, dtb_ref, alog_ref, cw_ref,
                   o_ref)

    @pl.when(par == 1)
    def _odd():
        @pl.when(j + 1 < B // 8)
        def _pre():
            for cp in _copies(j + 1, scr, sems):
                cp.start()
        for cp in _copies(j, scr1, sems1):
            cp.wait()
        _o_compute(scr1, mq_ref, b_ref, a_ref, dtb_ref, alog_ref, cw_ref,
                   o_ref)


def _o_compute(scr, mq_ref, b_ref, a_ref, dtb_ref, alog_ref, cw_ref, o_ref):
    x = _conv_x(scr[:, SSM:SSM + QKV],
                scr[:, SSM + QKV:SSM + 2 * QKV],
                scr[:, SSM + 2 * QKV:SSM + 3 * QKV],
                mq_ref[...], cw_ref)
    g, beta = _gate(a_ref[...], b_ref[...], dtb_ref[:], alog_ref[:])
    E = _seg_matrix()
    for hh in range(HV):
        h = hh // 2
        K = _l2n(x[:, (HK + h) * DK:(HK + h + 1) * DK])
        Q = _l2n(x[:, h * DK:(h + 1) * DK]) * (DK ** -0.5)
        V = x[:, (2 * HK + hh) * DV:(2 * HK + hh + 1) * DV]
        base = hh * HSZ
        Kt = jnp.concatenate([K] * NSEG, axis=1)
        Qt = jnp.concatenate([Q] * NSEG, axis=1)
        EGc = jnp.exp(g[:, hh:hh + 1])
        kvs, sqs = [], []
        for i in range(NSPAN):
            Sg = scr[:, base + i * SPAN:base + (i + 1) * SPAN] * EGc
            kvs.append(_seg_reduce(Sg * Kt, E))
            sqs.append(_seg_reduce(Sg * Qt, E))
        KVm = jnp.concatenate(kvs, axis=1)
        SQm = jnp.concatenate(sqs, axis=1)
        Dm = (V - KVm) * beta[:, hh:hh + 1]
        kq = jnp.sum(K * Q, axis=1, keepdims=True)
        o_ref[:, hh * DV:(hh + 1) * DV] = SQm + Dm * kq


def kernel(mixed_qkv, b, a, kv_cache_mem, conv_weights, dt_bias, alog,
           block_idx):
    f32 = jnp.float32
    cw = conv_weights.T
    dtb2 = dt_bias.reshape(1, HV)
    alog2 = alog.reshape(1, HV)
    tfb = jnp.full((NB,), -1, jnp.int32).at[block_idx].set(
        jnp.arange(B, dtype=jnp.int32))

    new_mem = pl.pallas_call(
        _mem_kernel,
        grid_spec=pltpu.PrefetchScalarGridSpec(
            num_scalar_prefetch=1,
            grid=(NB // 8,),
            in_specs=[
                pl.BlockSpec((8, BLOCK), lambda n, s: (n, 0)),
                pl.BlockSpec((B, QKV), lambda n, s: (0, 0)),
                pl.BlockSpec((B, HV), lambda n, s: (0, 0)),
                pl.BlockSpec((B, HV), lambda n, s: (0, 0)),
                pl.BlockSpec((1, HV), lambda n, s: (0, 0)),
                pl.BlockSpec((1, HV), lambda n, s: (0, 0)),
                pl.BlockSpec((KW, QKV), lambda n, s: (0, 0)),
            ],
            out_specs=pl.BlockSpec((8, BLOCK), lambda n, s: (n, 0)),
        ),
        out_shape=jax.ShapeDtypeStruct((NB, BLOCK), f32),
    )(tfb, kv_cache_mem, mixed_qkv, b, a, dtb2, alog2, cw)

    o_flat = pl.pallas_call(
        _o_kernel,
        grid_spec=pltpu.PrefetchScalarGridSpec(
            num_scalar_prefetch=1,
            grid=(B // 8,),
            in_specs=[
                pl.BlockSpec(memory_space=pl.ANY),
                pl.BlockSpec((8, QKV), lambda j, s: (j, 0)),
                pl.BlockSpec((8, HV), lambda j, s: (j, 0)),
                pl.BlockSpec((8, HV), lambda j, s: (j, 0)),
                pl.BlockSpec((1, HV), lambda j, s: (0, 0)),
                pl.BlockSpec((1, HV), lambda j, s: (0, 0)),
                pl.BlockSpec((KW, QKV), lambda j, s: (0, 0)),
            ],
            out_specs=pl.BlockSpec((8, HV * DV), lambda j, s: (j, 0)),
            scratch_shapes=[
                pltpu.VMEM((8, BLOCK), f32),
                pltpu.VMEM((8, BLOCK), f32),
                pltpu.SemaphoreType.DMA((8,)),
                pltpu.SemaphoreType.DMA((8,)),
            ],
        ),
        out_shape=jax.ShapeDtypeStruct((B, HV * DV), f32),
    )(block_idx, kv_cache_mem, mixed_qkv, b, a, dtb2, alog2, cw)

    return o_flat.reshape(B, HV, DV), new_mem
